# Initial kernel scaffold; baseline (speedup 1.0000x reference)
import jax
import jax.numpy as jnp
from jax.experimental import pallas as pl


def kernel(x, edge_index, edge_weight, W1, b1, W2, b2, Wfc, bfc):
    raise NotImplementedError("write your pallas kernel here")


# speedup vs baseline: 13.7234x; 13.7234x over previous
import dataclasses
import functools

import jax
import jax.numpy as jnp
from jax import lax
from jax.experimental import pallas as pl
from jax.experimental.pallas import tpu as pltpu
from jax.experimental.pallas import tpu_sc as plsc

N = 10000
E = 320000
D = 128
DO = 16
CH = 80
RC = E // CH
SUP = 8
NSUP = RC // SUP
SUPZ = 40
NSUPZ = N // SUPZ
NC, NS = 2, 16


def _fast_rsqrt(v):
    b = plsc.bitcast(v, jnp.int32)
    y = plsc.bitcast(jnp.int32(0x5F3759DF) - lax.shift_right_logical(b, 1),
                     jnp.float32)
    y = y * (1.5 - 0.5 * v * y * y)
    y = y * (1.5 - 0.5 * v * y * y)
    y = y * (1.5 - 0.5 * v * y * y)
    return y


def _make_sc_agg(first_layer):
    def body(*refs):
        if first_layer:
            (src_hbm, dst_hbm, ew_hbm, tab_hbm,
             agg_hbm, dinv_hbm,
             acc_sh, deg_sh, zbuf, z1buf, dinv_t,
             idx_s, idx_d, ew_s, rows_b) = refs
        else:
            (src_hbm, dst_hbm, ew_hbm, tab_hbm, dinv_in,
             agg_hbm,
             acc_sh, zbuf, dinv_t,
             idx_s, idx_d, ew_s, rows_b) = refs
        c = lax.axis_index("c")
        s = lax.axis_index("s")

        @pl.loop(0, SUPZ)
        def _(r):
            for j in range(0, D, 16):
                zbuf[r, pl.ds(j, 16)] = jnp.zeros((16,), jnp.float32)

        @pl.loop(s, NSUPZ, step=NS)
        def _(g):
            pltpu.sync_copy(zbuf, acc_sh.at[pl.ds(g * SUPZ, SUPZ)])

        if first_layer:
            @pl.loop(0, 1024, step=16)
            def _(i):
                z1buf[pl.ds(i, 16)] = jnp.zeros((16,), jnp.float32)

            @pl.when(s < 10)
            def _():
                pltpu.sync_copy(z1buf.at[pl.ds(0, 1000)],
                                deg_sh.at[pl.ds(s * 1000, 1000)])

        plsc.subcore_barrier()

        if first_layer:
            @pl.loop(s, NSUP, step=NS)
            def _(g):
                row0 = g * SUP
                pltpu.sync_copy(dst_hbm.at[pl.ds(row0, SUP)], idx_d)
                pltpu.sync_copy(ew_hbm.at[pl.ds(row0, SUP)], ew_s)

                @pl.loop(0, SUP)
                def _(j):
                    pltpu.sync_copy(ew_s.at[j], deg_sh.at[idx_d.at[j]],
                                    add=True)

            plsc.subcore_barrier()

            pltpu.sync_copy(deg_sh, dinv_t)

            @pl.loop(0, N, step=16)
            def _(i):
                v = dinv_t[pl.ds(i, 16)] + 1.0
                dinv_t[pl.ds(i, 16)] = _fast_rsqrt(v)

            @pl.when(jnp.logical_and(c == 0, s < 10))
            def _():
                pltpu.sync_copy(dinv_t.at[pl.ds(s * 1000, 1000)],
                                dinv_hbm.at[pl.ds(s * 1000, 1000)])
        else:
            pltpu.sync_copy(dinv_in, dinv_t)

        @pl.loop(s, NSUP // NC, step=NS)
        def _(g):
            row0 = c * (RC // NC) + g * SUP
            pltpu.sync_copy(src_hbm.at[pl.ds(row0, SUP)], idx_s)
            pltpu.sync_copy(dst_hbm.at[pl.ds(row0, SUP)], idx_d)
            pltpu.sync_copy(ew_hbm.at[pl.ds(row0, SUP)], ew_s)

            @pl.loop(0, SUP)
            def _(j):
                pltpu.sync_copy(tab_hbm.at[idx_s.at[j]], rows_b)

                @pl.loop(0, CH, step=16)
                def _(i):
                    idx = idx_s[j, pl.ds(i, 16)]
                    dv = plsc.load_gather(dinv_t, [idx])
                    wv = ew_s[j, pl.ds(i, 16)] * dv
                    for l in range(16):
                        w = wv[l]
                        for jj in range(0, D, 16):
                            rows_b[i + l, pl.ds(jj, 16)] = (
                                rows_b[i + l, pl.ds(jj, 16)] * w)

                pltpu.sync_copy(rows_b, acc_sh.at[idx_d.at[j]], add=True)

        plsc.subcore_barrier()

        @pl.loop(s, NSUPZ, step=NS)
        def _(g):
            r0 = g * SUPZ
            pltpu.sync_copy(acc_sh.at[pl.ds(r0, SUPZ)],
                            agg_hbm.at[c].at[pl.ds(r0, SUPZ)])

    if first_layer:
        out_type = (jax.ShapeDtypeStruct((NC, N, D), jnp.float32),
                    jax.ShapeDtypeStruct((N,), jnp.float32))
        scratch = [
            pltpu.VMEM_SHARED((N, D), jnp.float32),
            pltpu.VMEM_SHARED((N,), jnp.float32),
            pltpu.VMEM((SUPZ, D), jnp.float32),
            pltpu.VMEM((1024,), jnp.float32),
            pltpu.VMEM((N,), jnp.float32),
            pltpu.VMEM((SUP, CH), jnp.int32),
            pltpu.VMEM((SUP, CH), jnp.int32),
            pltpu.VMEM((SUP, CH), jnp.float32),
            pltpu.VMEM((CH, D), jnp.float32),
        ]
    else:
        out_type = jax.ShapeDtypeStruct((NC, N, D), jnp.float32)
        scratch = [
            pltpu.VMEM_SHARED((N, D), jnp.float32),
            pltpu.VMEM((SUPZ, D), jnp.float32),
            pltpu.VMEM((N,), jnp.float32),
            pltpu.VMEM((SUP, CH), jnp.int32),
            pltpu.VMEM((SUP, CH), jnp.int32),
            pltpu.VMEM((SUP, CH), jnp.float32),
            pltpu.VMEM((CH, D), jnp.float32),
        ]

    mesh = plsc.VectorSubcoreMesh(core_axis_name="c", subcore_axis_name="s",
                                  num_cores=NC, num_subcores=NS)
    cp = pltpu.CompilerParams()
    if "needs_layout_passes" in pltpu.CompilerParams.__dataclass_fields__:
        cp = dataclasses.replace(cp, needs_layout_passes=False)
    return pl.kernel(body, out_type=out_type, mesh=mesh, scratch_types=scratch,
                     compiler_params=cp)


def _tc_layer1(agg, x, dinv2d, W1, b1r):
    BR = 1000

    def body(a_ref, x_ref, dv_ref, w_ref, b_ref, o_ref):
        a = a_ref[0] + a_ref[1]
        dv = dv_ref[...]
        u = dv * a + (dv * dv) * x_ref[...]
        y = jnp.dot(u, w_ref[...], preferred_element_type=jnp.float32)
        o_ref[...] = jnp.maximum(y + b_ref[...], 0.0)

    return pl.pallas_call(
        body,
        grid=(N // BR,),
        in_specs=[
            pl.BlockSpec((NC, BR, D), lambda i: (0, i, 0)),
            pl.BlockSpec((BR, D), lambda i: (i, 0)),
            pl.BlockSpec((BR, 1), lambda i: (i, 0)),
            pl.BlockSpec((D, D), lambda i: (0, 0)),
            pl.BlockSpec((1, D), lambda i: (0, 0)),
        ],
        out_specs=pl.BlockSpec((BR, D), lambda i: (i, 0)),
        out_shape=jax.ShapeDtypeStruct((N, D), jnp.float32),
    )(agg, x, dinv2d, W1, b1r)


def _tc_layer2(agg, y1, dinv2d, W2, b2r, Wfc, bfcr):
    BR = 1000

    def body(a_ref, y_ref, dv_ref, w2_ref, b2_ref, wf_ref, bf_ref, o_ref):
        a = a_ref[0] + a_ref[1]
        dv = dv_ref[...]
        u = dv * a + (dv * dv) * y_ref[...]
        y2 = jnp.maximum(
            jnp.dot(u, w2_ref[...], preferred_element_type=jnp.float32)
            + b2_ref[...], 0.0)
        o_ref[...] = (jnp.dot(y2, wf_ref[...],
                              preferred_element_type=jnp.float32)
                      + bf_ref[...])

    return pl.pallas_call(
        body,
        grid=(N // BR,),
        in_specs=[
            pl.BlockSpec((NC, BR, D), lambda i: (0, i, 0)),
            pl.BlockSpec((BR, D), lambda i: (i, 0)),
            pl.BlockSpec((BR, 1), lambda i: (i, 0)),
            pl.BlockSpec((D, D), lambda i: (0, 0)),
            pl.BlockSpec((1, D), lambda i: (0, 0)),
            pl.BlockSpec((D, DO), lambda i: (0, 0)),
            pl.BlockSpec((1, DO), lambda i: (0, 0)),
        ],
        out_specs=pl.BlockSpec((BR, DO), lambda i: (i, 0)),
        out_shape=jax.ShapeDtypeStruct((N, DO), jnp.float32),
    )(agg, y1, dinv2d, W2, b2r, Wfc, bfcr)


@functools.lru_cache(maxsize=None)
def _sc_agg(first_layer):
    return _make_sc_agg(first_layer)


def kernel(x, edge_index, edge_weight, W1, b1, W2, b2, Wfc, bfc):
    src2 = edge_index[0].reshape(RC, CH)
    dst2 = edge_index[1].reshape(RC, CH)
    ew2 = edge_weight.reshape(RC, CH)

    agg1, dinv = _sc_agg(True)(src2, dst2, ew2, x)
    dinv2d = dinv.reshape(N, 1)
    y1 = _tc_layer1(agg1, x, dinv2d, W1, b1.reshape(1, D))
    agg2 = _sc_agg(False)(src2, dst2, ew2, y1, dinv)
    out = _tc_layer2(agg2, y1, dinv2d, W2, b2.reshape(1, D),
                     Wfc, bfc.reshape(1, DO))
    return out


# speedup vs baseline: 18.9526x; 1.3810x over previous
import dataclasses
import functools

import jax
import jax.numpy as jnp
from jax import lax
from jax.experimental import pallas as pl
from jax.experimental.pallas import tpu as pltpu
from jax.experimental.pallas import tpu_sc as plsc

N = 10000
E = 320000
D = 128
DO = 16
CH = 80
RC = E // CH
SUP = 8
SUPZ = 40
NSUPZ = N // SUPZ
NC, NS = 2, 16
NW = NC * NS
NSUP_AGG = RC // NC // SUP
NSUP_DEG = RC // SUP


def _fast_rsqrt(v):
    b = plsc.bitcast(v, jnp.int32)
    y = plsc.bitcast(jnp.int32(0x5F3759DF) - lax.shift_right_logical(b, 1),
                     jnp.float32)
    y = y * (1.5 - 0.5 * v * y * y)
    y = y * (1.5 - 0.5 * v * y * y)
    y = y * (1.5 - 0.5 * v * y * y)
    return y


def _make_sc_agg(first_layer):
    def body(*refs):
        if first_layer:
            (src_hbm, dst_hbm, ew_hbm, tab_hbm,
             agg_hbm, dinv_hbm,
             acc_sh, deg_sh, zbuf, z1buf, dinv_t,
             stg_src, stg_dst, stg_ew, rb0, rb1,
             sz, si, sg0, sg1, ss0, ss1, sdeg) = refs
        else:
            (src_hbm, dst_hbm, ew_hbm, tab_hbm, dinv_in,
             agg_hbm,
             acc_sh, zbuf, dinv_t,
             stg_src, stg_dst, stg_ew, rb0, rb1,
             sz, si, sg0, sg1, ss0, ss1) = refs
        c = lax.axis_index("c")
        s = lax.axis_index("s")

        @pl.loop(0, SUPZ)
        def _(r):
            for j in range(0, D, 16):
                zbuf[r, pl.ds(j, 16)] = jnp.zeros((16,), jnp.float32)

        @pl.loop(s, NSUPZ, step=NS)
        def _(g):
            pltpu.async_copy(zbuf, acc_sh.at[pl.ds(g * SUPZ, SUPZ)], sz)

        if first_layer:
            @pl.loop(0, 1024, step=16)
            def _(i):
                z1buf[pl.ds(i, 16)] = jnp.zeros((16,), jnp.float32)

            @pl.when(s < 10)
            def _():
                pltpu.sync_copy(z1buf.at[pl.ds(0, 1000)],
                                deg_sh.at[pl.ds(s * 1000, 1000)])

        @pl.loop(s, NSUPZ, step=NS)
        def _(g):
            pltpu.make_async_copy(zbuf, acc_sh.at[pl.ds(0, SUPZ)], sz).wait()

        plsc.subcore_barrier()

        if first_layer:
            nsd = (NSUP_DEG + NS - 1 - s) // NS

            pltpu.async_copy(dst_hbm.at[pl.ds(s * SUP, SUP)], stg_dst.at[0],
                             si)
            pltpu.async_copy(ew_hbm.at[pl.ds(s * SUP, SUP)], stg_ew.at[0], si)

            @pl.loop(0, nsd)
            def _(t):
                par = t % 2

                @pl.when(t > 0)
                def _():
                    @pl.loop(0, SUP)
                    def _(j):
                        pltpu.make_async_copy(
                            stg_ew.at[0].at[0],
                            deg_sh.at[stg_dst.at[0].at[0]], sdeg).wait()

                pltpu.make_async_copy(dst_hbm.at[pl.ds(0, SUP)],
                                      stg_dst.at[0], si).wait()
                pltpu.make_async_copy(ew_hbm.at[pl.ds(0, SUP)],
                                      stg_ew.at[0], si).wait()

                @pl.when(t + 1 < nsd)
                def _():
                    r1 = (s + NS * (t + 1)) * SUP
                    pltpu.async_copy(dst_hbm.at[pl.ds(r1, SUP)],
                                     stg_dst.at[1 - par], si)
                    pltpu.async_copy(ew_hbm.at[pl.ds(r1, SUP)],
                                     stg_ew.at[1 - par], si)

                @pl.loop(0, SUP)
                def _(j):
                    pltpu.async_copy(stg_ew.at[par].at[j],
                                     deg_sh.at[stg_dst.at[par].at[j]], sdeg,
                                     add=True)

            @pl.loop(0, SUP)
            def _(j):
                pltpu.make_async_copy(stg_ew.at[0].at[0],
                                      deg_sh.at[stg_dst.at[0].at[0]],
                                      sdeg).wait()

            plsc.subcore_barrier()

            pltpu.sync_copy(deg_sh, dinv_t)

            @pl.loop(0, N, step=16)
            def _(i):
                v = dinv_t[pl.ds(i, 16)] + 1.0
                dinv_t[pl.ds(i, 16)] = _fast_rsqrt(v)

            @pl.when(jnp.logical_and(c == 0, s < 10))
            def _():
                pltpu.sync_copy(dinv_t.at[pl.ds(s * 1000, 1000)],
                                dinv_hbm.at[pl.ds(s * 1000, 1000)])
        else:
            pltpu.sync_copy(dinv_in, dinv_t)

        nsa = (NSUP_AGG + NS - 1 - s) // NS
        base = c * (RC // NC)

        def scale_rows(par, q, rb):
            @pl.loop(0, CH, step=16)
            def _(i):
                idx = stg_src[par, q, pl.ds(i, 16)]
                dv = plsc.load_gather(dinv_t, [idx])
                wv = stg_ew[par, q, pl.ds(i, 16)] * dv
                for l in range(16):
                    wl = wv[l]
                    for jj in range(0, D, 16):
                        rb[i + l, pl.ds(jj, 16)] = (
                            rb[i + l, pl.ds(jj, 16)] * wl)

        def scatter_drain(rb, ss):
            pltpu.make_async_copy(rb, acc_sh.at[stg_dst.at[0].at[0]],
                                  ss).wait()

        def chunk(t, par, q, rb_m, sg_m, ss_m, rb_o, sg_o, ss_o):
            pltpu.make_async_copy(tab_hbm.at[stg_src.at[par].at[q]], rb_m,
                                  sg_m).wait()
            scale_rows(par, q, rb_m)
            pltpu.async_copy(rb_m, acc_sh.at[stg_dst.at[par].at[q]], ss_m,
                             add=True)
            if q == 0:
                @pl.when(t > 0)
                def _():
                    scatter_drain(rb_o, ss_o)
            else:
                scatter_drain(rb_o, ss_o)
            if q + 1 < SUP:
                pltpu.async_copy(tab_hbm.at[stg_src.at[par].at[q + 1]], rb_o,
                                 sg_o)
            else:
                @pl.when(t + 1 < nsa)
                def _():
                    pltpu.async_copy(tab_hbm.at[stg_src.at[1 - par].at[0]],
                                     rb_o, sg_o)

        def stage_agg(t, par, sem):
            r = base + (s + NS * t) * SUP
            pltpu.async_copy(src_hbm.at[pl.ds(r, SUP)], stg_src.at[par], sem)
            pltpu.async_copy(dst_hbm.at[pl.ds(r, SUP)], stg_dst.at[par], sem)
            pltpu.async_copy(ew_hbm.at[pl.ds(r, SUP)], stg_ew.at[par], sem)

        def drain_stage(sem):
            pltpu.make_async_copy(src_hbm.at[pl.ds(0, SUP)], stg_src.at[0],
                                  sem).wait()
            pltpu.make_async_copy(dst_hbm.at[pl.ds(0, SUP)], stg_dst.at[0],
                                  sem).wait()
            pltpu.make_async_copy(ew_hbm.at[pl.ds(0, SUP)], stg_ew.at[0],
                                  sem).wait()

        stage_agg(0, 0, si)
        drain_stage(si)
        pltpu.async_copy(tab_hbm.at[stg_src.at[0].at[0]], rb0, sg0)

        @pl.loop(0, nsa)
        def _(t):
            par = t % 2
            chunk(t, par, 0, rb0, sg0, ss0, rb1, sg1, ss1)

            @pl.when(t + 1 < nsa)
            def _():
                stage_agg(t + 1, 1 - par, si)

            chunk(t, par, 1, rb1, sg1, ss1, rb0, sg0, ss0)
            chunk(t, par, 2, rb0, sg0, ss0, rb1, sg1, ss1)
            chunk(t, par, 3, rb1, sg1, ss1, rb0, sg0, ss0)
            chunk(t, par, 4, rb0, sg0, ss0, rb1, sg1, ss1)
            chunk(t, par, 5, rb1, sg1, ss1, rb0, sg0, ss0)
            chunk(t, par, 6, rb0, sg0, ss0, rb1, sg1, ss1)

            @pl.when(t + 1 < nsa)
            def _():
                drain_stage(si)

            chunk(t, par, 7, rb1, sg1, ss1, rb0, sg0, ss0)

        scatter_drain(rb1, ss1)

        plsc.subcore_barrier()

        @pl.loop(s, NSUPZ, step=NS)
        def _(g):
            r0 = g * SUPZ
            pltpu.async_copy(acc_sh.at[pl.ds(r0, SUPZ)],
                             agg_hbm.at[c].at[pl.ds(r0, SUPZ)], sz)

        @pl.loop(s, NSUPZ, step=NS)
        def _(g):
            pltpu.make_async_copy(acc_sh.at[pl.ds(0, SUPZ)],
                                  agg_hbm.at[c].at[pl.ds(0, SUPZ)], sz).wait()

    if first_layer:
        out_type = (jax.ShapeDtypeStruct((NC, N, D), jnp.float32),
                    jax.ShapeDtypeStruct((N,), jnp.float32))
    else:
        out_type = jax.ShapeDtypeStruct((NC, N, D), jnp.float32)

    scratch = [
        pltpu.VMEM_SHARED((N, D), jnp.float32),
    ]
    if first_layer:
        scratch += [
            pltpu.VMEM_SHARED((N,), jnp.float32),
        ]
    scratch += [
        pltpu.VMEM((SUPZ, D), jnp.float32),
    ]
    if first_layer:
        scratch += [
            pltpu.VMEM((1024,), jnp.float32),
        ]
    scratch += [
        pltpu.VMEM((N,), jnp.float32),
        pltpu.VMEM((2, SUP, CH), jnp.int32),
        pltpu.VMEM((2, SUP, CH), jnp.int32),
        pltpu.VMEM((2, SUP, CH), jnp.float32),
        pltpu.VMEM((CH, D), jnp.float32),
        pltpu.VMEM((CH, D), jnp.float32),
        pltpu.SemaphoreType.DMA,
        pltpu.SemaphoreType.DMA,
        pltpu.SemaphoreType.DMA,
        pltpu.SemaphoreType.DMA,
        pltpu.SemaphoreType.DMA,
        pltpu.SemaphoreType.DMA,
    ]
    if first_layer:
        scratch += [
            pltpu.SemaphoreType.DMA,
        ]

    mesh = plsc.VectorSubcoreMesh(core_axis_name="c", subcore_axis_name="s",
                                  num_cores=NC, num_subcores=NS)
    cp = pltpu.CompilerParams()
    if "needs_layout_passes" in pltpu.CompilerParams.__dataclass_fields__:
        cp = dataclasses.replace(cp, needs_layout_passes=False)
    return pl.kernel(body, out_type=out_type, mesh=mesh, scratch_types=scratch,
                     compiler_params=cp)


def _tc_layer1(agg, x, dinv2d, W1, b1r):
    BR = 1000

    def body(a_ref, x_ref, dv_ref, w_ref, b_ref, o_ref):
        a = a_ref[0] + a_ref[1]
        dv = dv_ref[...]
        u = dv * a + (dv * dv) * x_ref[...]
        y = jnp.dot(u, w_ref[...], preferred_element_type=jnp.float32)
        o_ref[...] = jnp.maximum(y + b_ref[...], 0.0)

    return pl.pallas_call(
        body,
        grid=(N // BR,),
        in_specs=[
            pl.BlockSpec((NC, BR, D), lambda i: (0, i, 0)),
            pl.BlockSpec((BR, D), lambda i: (i, 0)),
            pl.BlockSpec((BR, 1), lambda i: (i, 0)),
            pl.BlockSpec((D, D), lambda i: (0, 0)),
            pl.BlockSpec((1, D), lambda i: (0, 0)),
        ],
        out_specs=pl.BlockSpec((BR, D), lambda i: (i, 0)),
        out_shape=jax.ShapeDtypeStruct((N, D), jnp.float32),
    )(agg, x, dinv2d, W1, b1r)


def _tc_layer2(agg, y1, dinv2d, W2, b2r, Wfc, bfcr):
    BR = 1000

    def body(a_ref, y_ref, dv_ref, w2_ref, b2_ref, wf_ref, bf_ref, o_ref):
        a = a_ref[0] + a_ref[1]
        dv = dv_ref[...]
        u = dv * a + (dv * dv) * y_ref[...]
        y2 = jnp.maximum(
            jnp.dot(u, w2_ref[...], preferred_element_type=jnp.float32)
            + b2_ref[...], 0.0)
        o_ref[...] = (jnp.dot(y2, wf_ref[...],
                              preferred_element_type=jnp.float32)
                      + bf_ref[...])

    return pl.pallas_call(
        body,
        grid=(N // BR,),
        in_specs=[
            pl.BlockSpec((NC, BR, D), lambda i: (0, i, 0)),
            pl.BlockSpec((BR, D), lambda i: (i, 0)),
            pl.BlockSpec((BR, 1), lambda i: (i, 0)),
            pl.BlockSpec((D, D), lambda i: (0, 0)),
            pl.BlockSpec((1, D), lambda i: (0, 0)),
            pl.BlockSpec((D, DO), lambda i: (0, 0)),
            pl.BlockSpec((1, DO), lambda i: (0, 0)),
        ],
        out_specs=pl.BlockSpec((BR, DO), lambda i: (i, 0)),
        out_shape=jax.ShapeDtypeStruct((N, DO), jnp.float32),
    )(agg, y1, dinv2d, W2, b2r, Wfc, bfcr)


@functools.lru_cache(maxsize=None)
def _sc_agg(first_layer):
    return _make_sc_agg(first_layer)


def kernel(x, edge_index, edge_weight, W1, b1, W2, b2, Wfc, bfc):
    src2 = edge_index[0].reshape(RC, CH)
    dst2 = edge_index[1].reshape(RC, CH)
    ew2 = edge_weight.reshape(RC, CH)

    agg1, dinv = _sc_agg(True)(src2, dst2, ew2, x)
    dinv2d = dinv.reshape(N, 1)
    y1 = _tc_layer1(agg1, x, dinv2d, W1, b1.reshape(1, D))
    agg2 = _sc_agg(False)(src2, dst2, ew2, y1, dinv)
    out = _tc_layer2(agg2, y1, dinv2d, W2, b2.reshape(1, D),
                     Wfc, bfc.reshape(1, DO))
    return out


# speedup vs baseline: 23.6590x; 1.2483x over previous
import dataclasses
import functools

import jax
import jax.numpy as jnp
from jax import lax
from jax.experimental import pallas as pl
from jax.experimental.pallas import tpu as pltpu
from jax.experimental.pallas import tpu_sc as plsc

N = 10000
E = 320000
D = 128
DO = 16
CH = 80
RC = E // CH
SUP = 8
SUPZ = 40
NSUPZ = N // SUPZ
NC, NS = 2, 16
NW = NC * NS
NSUP_AGG = RC // NC // SUP
NSUP_DEG = RC // SUP


def _fast_rsqrt(v):
    b = plsc.bitcast(v, jnp.int32)
    y = plsc.bitcast(jnp.int32(0x5F3759DF) - lax.shift_right_logical(b, 1),
                     jnp.float32)
    y = y * (1.5 - 0.5 * v * y * y)
    y = y * (1.5 - 0.5 * v * y * y)
    y = y * (1.5 - 0.5 * v * y * y)
    return y


def _make_sc_agg(first_layer):
    def body(*refs):
        if first_layer:
            (src_hbm, dst_hbm, ew_hbm, tab_hbm,
             agg_hbm, dinv_hbm,
             acc_sh, deg_sh, zbuf, z1buf, dinv_t,
             stg_src, stg_dst, stg_ew, rb0, rb1,
             sz, si, sg0, sg1, ss0, ss1, sdeg) = refs
        else:
            (src_hbm, dst_hbm, ew_hbm, tab_hbm, dinv_in,
             agg_hbm,
             acc_sh, zbuf, dinv_t,
             stg_src, stg_dst, stg_ew, rb0, rb1,
             sz, si, sg0, sg1, ss0, ss1) = refs
        c = lax.axis_index("c")
        s = lax.axis_index("s")

        @pl.loop(0, SUPZ)
        def _(r):
            for j in range(0, D, 16):
                zbuf[r, pl.ds(j, 16)] = jnp.zeros((16,), jnp.float32)

        @pl.loop(s, NSUPZ, step=NS)
        def _(g):
            pltpu.async_copy(zbuf, acc_sh.at[pl.ds(g * SUPZ, SUPZ)], sz)

        if first_layer:
            @pl.loop(0, 1024, step=16)
            def _(i):
                z1buf[pl.ds(i, 16)] = jnp.zeros((16,), jnp.float32)

            @pl.when(s < 10)
            def _():
                pltpu.sync_copy(z1buf.at[pl.ds(0, 1000)],
                                deg_sh.at[pl.ds(s * 1000, 1000)])

        @pl.loop(s, NSUPZ, step=NS)
        def _(g):
            pltpu.make_async_copy(zbuf, acc_sh.at[pl.ds(0, SUPZ)], sz).wait()

        plsc.subcore_barrier()

        if first_layer:
            nsd = (NSUP_DEG + NS - 1 - s) // NS

            pltpu.async_copy(dst_hbm.at[pl.ds(s * SUP, SUP)], stg_dst.at[0],
                             si)
            pltpu.async_copy(ew_hbm.at[pl.ds(s * SUP, SUP)], stg_ew.at[0], si)

            @pl.loop(0, nsd)
            def _(t):
                par = t % 2

                @pl.when(t > 0)
                def _():
                    @pl.loop(0, SUP)
                    def _(j):
                        pltpu.make_async_copy(
                            stg_ew.at[0].at[0],
                            deg_sh.at[stg_dst.at[0].at[0]], sdeg).wait()

                pltpu.make_async_copy(dst_hbm.at[pl.ds(0, SUP)],
                                      stg_dst.at[0], si).wait()
                pltpu.make_async_copy(ew_hbm.at[pl.ds(0, SUP)],
                                      stg_ew.at[0], si).wait()

                @pl.when(t + 1 < nsd)
                def _():
                    r1 = (s + NS * (t + 1)) * SUP
                    pltpu.async_copy(dst_hbm.at[pl.ds(r1, SUP)],
                                     stg_dst.at[1 - par], si)
                    pltpu.async_copy(ew_hbm.at[pl.ds(r1, SUP)],
                                     stg_ew.at[1 - par], si)

                @pl.loop(0, SUP)
                def _(j):
                    pltpu.async_copy(stg_ew.at[par].at[j],
                                     deg_sh.at[stg_dst.at[par].at[j]], sdeg,
                                     add=True)

            @pl.loop(0, SUP)
            def _(j):
                pltpu.make_async_copy(stg_ew.at[0].at[0],
                                      deg_sh.at[stg_dst.at[0].at[0]],
                                      sdeg).wait()

            plsc.subcore_barrier()

            pltpu.sync_copy(deg_sh, dinv_t)

            @pl.loop(0, N, step=16)
            def _(i):
                v = dinv_t[pl.ds(i, 16)] + 1.0
                dinv_t[pl.ds(i, 16)] = _fast_rsqrt(v)

            @pl.when(jnp.logical_and(c == 0, s < 10))
            def _():
                pltpu.sync_copy(dinv_t.at[pl.ds(s * 1000, 1000)],
                                dinv_hbm.at[pl.ds(s * 1000, 1000)])
        else:
            pltpu.sync_copy(dinv_in, dinv_t)

        nsa = (NSUP_AGG + NS - 1 - s) // NS
        base = c * (RC // NC)

        def scale_rows(par, q, rb):
            @pl.loop(0, CH, step=16)
            def _(i):
                idx = stg_src[par, q, pl.ds(i, 16)]
                dv = plsc.load_gather(dinv_t, [idx])
                wv = stg_ew[par, q, pl.ds(i, 16)] * dv
                for l in range(16):
                    wl = wv[l]
                    for jj in range(0, D, 16):
                        rb[i + l, pl.ds(jj, 16)] = (
                            rb[i + l, pl.ds(jj, 16)] * wl)

        def scatter_drain(rb, ss):
            pltpu.make_async_copy(rb, acc_sh.at[stg_dst.at[0].at[0]],
                                  ss).wait()

        def chunk(t, par, q, rb_m, sg_m, ss_m, rb_o, sg_o, ss_o):
            pltpu.make_async_copy(tab_hbm.at[stg_src.at[par].at[q]], rb_m,
                                  sg_m).wait()
            if q == 0:
                @pl.when(t > 0)
                def _():
                    scatter_drain(rb_o, ss_o)
            else:
                scatter_drain(rb_o, ss_o)
            if q + 1 < SUP:
                pltpu.async_copy(tab_hbm.at[stg_src.at[par].at[q + 1]], rb_o,
                                 sg_o)
            else:
                @pl.when(t + 1 < nsa)
                def _():
                    pltpu.async_copy(tab_hbm.at[stg_src.at[1 - par].at[0]],
                                     rb_o, sg_o)
            scale_rows(par, q, rb_m)
            pltpu.async_copy(rb_m, acc_sh.at[stg_dst.at[par].at[q]], ss_m,
                             add=True)

        def stage_agg(t, par, sem):
            r = base + (s + NS * t) * SUP
            pltpu.async_copy(src_hbm.at[pl.ds(r, SUP)], stg_src.at[par], sem)
            pltpu.async_copy(dst_hbm.at[pl.ds(r, SUP)], stg_dst.at[par], sem)
            pltpu.async_copy(ew_hbm.at[pl.ds(r, SUP)], stg_ew.at[par], sem)

        def drain_stage(sem):
            pltpu.make_async_copy(src_hbm.at[pl.ds(0, SUP)], stg_src.at[0],
                                  sem).wait()
            pltpu.make_async_copy(dst_hbm.at[pl.ds(0, SUP)], stg_dst.at[0],
                                  sem).wait()
            pltpu.make_async_copy(ew_hbm.at[pl.ds(0, SUP)], stg_ew.at[0],
                                  sem).wait()

        stage_agg(0, 0, si)
        drain_stage(si)
        pltpu.async_copy(tab_hbm.at[stg_src.at[0].at[0]], rb0, sg0)

        @pl.loop(0, nsa)
        def _(t):
            par = t % 2
            chunk(t, par, 0, rb0, sg0, ss0, rb1, sg1, ss1)

            @pl.when(t + 1 < nsa)
            def _():
                stage_agg(t + 1, 1 - par, si)

            chunk(t, par, 1, rb1, sg1, ss1, rb0, sg0, ss0)
            chunk(t, par, 2, rb0, sg0, ss0, rb1, sg1, ss1)
            chunk(t, par, 3, rb1, sg1, ss1, rb0, sg0, ss0)
            chunk(t, par, 4, rb0, sg0, ss0, rb1, sg1, ss1)
            chunk(t, par, 5, rb1, sg1, ss1, rb0, sg0, ss0)
            chunk(t, par, 6, rb0, sg0, ss0, rb1, sg1, ss1)

            @pl.when(t + 1 < nsa)
            def _():
                drain_stage(si)

            chunk(t, par, 7, rb1, sg1, ss1, rb0, sg0, ss0)

        scatter_drain(rb1, ss1)

        plsc.subcore_barrier()

        @pl.loop(s, NSUPZ, step=NS)
        def _(g):
            r0 = g * SUPZ
            pltpu.async_copy(acc_sh.at[pl.ds(r0, SUPZ)],
                             agg_hbm.at[c].at[pl.ds(r0, SUPZ)], sz)

        @pl.loop(s, NSUPZ, step=NS)
        def _(g):
            pltpu.make_async_copy(acc_sh.at[pl.ds(0, SUPZ)],
                                  agg_hbm.at[c].at[pl.ds(0, SUPZ)], sz).wait()

    if first_layer:
        out_type = (jax.ShapeDtypeStruct((NC, N, D), jnp.float32),
                    jax.ShapeDtypeStruct((N,), jnp.float32))
    else:
        out_type = jax.ShapeDtypeStruct((NC, N, D), jnp.float32)

    scratch = [
        pltpu.VMEM_SHARED((N, D), jnp.float32),
    ]
    if first_layer:
        scratch += [
            pltpu.VMEM_SHARED((N,), jnp.float32),
        ]
    scratch += [
        pltpu.VMEM((SUPZ, D), jnp.float32),
    ]
    if first_layer:
        scratch += [
            pltpu.VMEM((1024,), jnp.float32),
        ]
    scratch += [
        pltpu.VMEM((N,), jnp.float32),
        pltpu.VMEM((2, SUP, CH), jnp.int32),
        pltpu.VMEM((2, SUP, CH), jnp.int32),
        pltpu.VMEM((2, SUP, CH), jnp.float32),
        pltpu.VMEM((CH, D), jnp.float32),
        pltpu.VMEM((CH, D), jnp.float32),
        pltpu.SemaphoreType.DMA,
        pltpu.SemaphoreType.DMA,
        pltpu.SemaphoreType.DMA,
        pltpu.SemaphoreType.DMA,
        pltpu.SemaphoreType.DMA,
        pltpu.SemaphoreType.DMA,
    ]
    if first_layer:
        scratch += [
            pltpu.SemaphoreType.DMA,
        ]

    mesh = plsc.VectorSubcoreMesh(core_axis_name="c", subcore_axis_name="s",
                                  num_cores=NC, num_subcores=NS)
    cp = pltpu.CompilerParams()
    if "needs_layout_passes" in pltpu.CompilerParams.__dataclass_fields__:
        cp = dataclasses.replace(cp, needs_layout_passes=False)
    return pl.kernel(body, out_type=out_type, mesh=mesh, scratch_types=scratch,
                     compiler_params=cp)


def _tc_layer1(agg, x, dinv2d, W1, b1r):
    BR = 1000

    def body(a_ref, x_ref, dv_ref, w_ref, b_ref, o_ref):
        a = a_ref[0] + a_ref[1]
        dv = dv_ref[...]
        u = dv * a + (dv * dv) * x_ref[...]
        y = jnp.dot(u, w_ref[...], preferred_element_type=jnp.float32)
        o_ref[...] = jnp.maximum(y + b_ref[...], 0.0)

    return pl.pallas_call(
        body,
        grid=(N // BR,),
        in_specs=[
            pl.BlockSpec((NC, BR, D), lambda i: (0, i, 0)),
            pl.BlockSpec((BR, D), lambda i: (i, 0)),
            pl.BlockSpec((BR, 1), lambda i: (i, 0)),
            pl.BlockSpec((D, D), lambda i: (0, 0)),
            pl.BlockSpec((1, D), lambda i: (0, 0)),
        ],
        out_specs=pl.BlockSpec((BR, D), lambda i: (i, 0)),
        out_shape=jax.ShapeDtypeStruct((N, D), jnp.float32),
    )(agg, x, dinv2d, W1, b1r)


def _tc_layer2(agg, y1, dinv2d, W2, b2r, Wfc, bfcr):
    BR = 1000

    def body(a_ref, y_ref, dv_ref, w2_ref, b2_ref, wf_ref, bf_ref, o_ref):
        a = a_ref[0] + a_ref[1]
        dv = dv_ref[...]
        u = dv * a + (dv * dv) * y_ref[...]
        y2 = jnp.maximum(
            jnp.dot(u, w2_ref[...], preferred_element_type=jnp.float32)
            + b2_ref[...], 0.0)
        o_ref[...] = (jnp.dot(y2, wf_ref[...],
                              preferred_element_type=jnp.float32)
                      + bf_ref[...])

    return pl.pallas_call(
        body,
        grid=(N // BR,),
        in_specs=[
            pl.BlockSpec((NC, BR, D), lambda i: (0, i, 0)),
            pl.BlockSpec((BR, D), lambda i: (i, 0)),
            pl.BlockSpec((BR, 1), lambda i: (i, 0)),
            pl.BlockSpec((D, D), lambda i: (0, 0)),
            pl.BlockSpec((1, D), lambda i: (0, 0)),
            pl.BlockSpec((D, DO), lambda i: (0, 0)),
            pl.BlockSpec((1, DO), lambda i: (0, 0)),
        ],
        out_specs=pl.BlockSpec((BR, DO), lambda i: (i, 0)),
        out_shape=jax.ShapeDtypeStruct((N, DO), jnp.float32),
    )(agg, y1, dinv2d, W2, b2r, Wfc, bfcr)


@functools.lru_cache(maxsize=None)
def _sc_agg(first_layer):
    return _make_sc_agg(first_layer)


def kernel(x, edge_index, edge_weight, W1, b1, W2, b2, Wfc, bfc):
    src2 = edge_index[0].reshape(RC, CH)
    dst2 = edge_index[1].reshape(RC, CH)
    ew2 = edge_weight.reshape(RC, CH)

    agg1, dinv = _sc_agg(True)(src2, dst2, ew2, x)
    dinv2d = dinv.reshape(N, 1)
    y1 = _tc_layer1(agg1, x, dinv2d, W1, b1.reshape(1, D))
    agg2 = _sc_agg(False)(src2, dst2, ew2, y1, dinv)
    out = _tc_layer2(agg2, y1, dinv2d, W2, b2.reshape(1, D),
                     Wfc, bfc.reshape(1, DO))
    return out


# speedup vs baseline: 25.2414x; 1.0669x over previous
import dataclasses
import functools

import jax
import jax.numpy as jnp
from jax import lax
from jax.experimental import pallas as pl
from jax.experimental.pallas import tpu as pltpu
from jax.experimental.pallas import tpu_sc as plsc

N = 10000
E = 320000
D = 128
DH = 64
DO = 16
CH = 80
RC = E // CH
SUP = 8
NSUP = RC // SUP
SUPZ = 40
NSUPZ = N // SUPZ
NC, NS = 2, 16


def _fast_rsqrt(v):
    b = plsc.bitcast(v, jnp.int32)
    y = plsc.bitcast(jnp.int32(0x5F3759DF) - lax.shift_right_logical(b, 1),
                     jnp.float32)
    y = y * (1.5 - 0.5 * v * y * y)
    y = y * (1.5 - 0.5 * v * y * y)
    y = y * (1.5 - 0.5 * v * y * y)
    return y


def _make_sc_agg(first_layer):
    def body(*refs):
        if first_layer:
            (src_hbm, dst_hbm, ew_hbm, tabs_hbm,
             agg_hbm, dinv_hbm,
             acc_sh, deg_sh, zbuf, z1buf, dinv_t,
             stg_src, stg_dst, stg_ew, rb0, rb1, rb2, rb3,
             sz, si, sg0, sg1, sg2, sg3, ss0, ss1, ss2, ss3, sdeg) = refs
        else:
            (src_hbm, dst_hbm, ew_hbm, tabs_hbm, dinv_in,
             agg_hbm,
             acc_sh, zbuf, dinv_t,
             stg_src, stg_dst, stg_ew, rb0, rb1, rb2, rb3,
             sz, si, sg0, sg1, sg2, sg3, ss0, ss1, ss2, ss3) = refs
        c = lax.axis_index("c")
        s = lax.axis_index("s")
        rb = (rb0, rb1, rb2, rb3)
        sg = (sg0, sg1, sg2, sg3)
        ss = (ss0, ss1, ss2, ss3)
        tab = tabs_hbm.at[c]

        nsa = (NSUP + NS - 1 - s) // NS

        @pl.loop(0, SUPZ)
        def _(r):
            for j in range(0, DH, 16):
                zbuf[r, pl.ds(j, 16)] = jnp.zeros((16,), jnp.float32)

        @pl.loop(s, NSUPZ, step=NS)
        def _(g):
            pltpu.async_copy(zbuf, acc_sh.at[pl.ds(g * SUPZ, SUPZ)], sz)

        if first_layer:
            @pl.loop(0, 1024, step=16)
            def _(i):
                z1buf[pl.ds(i, 16)] = jnp.zeros((16,), jnp.float32)

            @pl.when(s < 10)
            def _():
                pltpu.sync_copy(z1buf.at[pl.ds(0, 1000)],
                                deg_sh.at[pl.ds(s * 1000, 1000)])

        @pl.loop(s, NSUPZ, step=NS)
        def _(g):
            pltpu.make_async_copy(zbuf, acc_sh.at[pl.ds(0, SUPZ)], sz).wait()

        plsc.subcore_barrier()

        if first_layer:
            pltpu.async_copy(dst_hbm.at[pl.ds(s * SUP, SUP)], stg_dst.at[0],
                             si)
            pltpu.async_copy(ew_hbm.at[pl.ds(s * SUP, SUP)], stg_ew.at[0], si)

            @pl.loop(0, nsa)
            def _(t):
                par = t % 2

                @pl.when(t > 0)
                def _():
                    @pl.loop(0, SUP)
                    def _(j):
                        pltpu.make_async_copy(
                            stg_ew.at[0].at[0],
                            deg_sh.at[stg_dst.at[0].at[0]], sdeg).wait()

                pltpu.make_async_copy(dst_hbm.at[pl.ds(0, SUP)],
                                      stg_dst.at[0], si).wait()
                pltpu.make_async_copy(ew_hbm.at[pl.ds(0, SUP)],
                                      stg_ew.at[0], si).wait()

                @pl.when(t + 1 < nsa)
                def _():
                    r1 = (s + NS * (t + 1)) * SUP
                    pltpu.async_copy(dst_hbm.at[pl.ds(r1, SUP)],
                                     stg_dst.at[1 - par], si)
                    pltpu.async_copy(ew_hbm.at[pl.ds(r1, SUP)],
                                     stg_ew.at[1 - par], si)

                @pl.loop(0, SUP)
                def _(j):
                    pltpu.async_copy(stg_ew.at[par].at[j],
                                     deg_sh.at[stg_dst.at[par].at[j]], sdeg,
                                     add=True)

            @pl.loop(0, SUP)
            def _(j):
                pltpu.make_async_copy(stg_ew.at[0].at[0],
                                      deg_sh.at[stg_dst.at[0].at[0]],
                                      sdeg).wait()

            plsc.subcore_barrier()

            pltpu.sync_copy(deg_sh, dinv_t)

            @pl.loop(0, N, step=16)
            def _(i):
                v = dinv_t[pl.ds(i, 16)] + 1.0
                dinv_t[pl.ds(i, 16)] = _fast_rsqrt(v)

            @pl.when(jnp.logical_and(c == 0, s < 10))
            def _():
                pltpu.sync_copy(dinv_t.at[pl.ds(s * 1000, 1000)],
                                dinv_hbm.at[pl.ds(s * 1000, 1000)])
        else:
            pltpu.sync_copy(dinv_in, dinv_t)

        def scale_rows(par, q, rbm):
            @pl.loop(0, CH, step=16)
            def _(i):
                idx = stg_src[par, q, pl.ds(i, 16)]
                dv = plsc.load_gather(dinv_t, [idx])
                wv = stg_ew[par, q, pl.ds(i, 16)] * dv
                for l in range(16):
                    wl = wv[l]
                    for jj in range(0, DH, 16):
                        rbm[i + l, pl.ds(jj, 16)] = (
                            rbm[i + l, pl.ds(jj, 16)] * wl)

        def scatter_drain(m):
            pltpu.make_async_copy(rb[m], acc_sh.at[stg_dst.at[0].at[0]],
                                  ss[m]).wait()

        def chunk(t, par, q):
            m = q % 4
            m2 = (q + 2) % 4
            pltpu.make_async_copy(tab.at[stg_src.at[par].at[q]], rb[m],
                                  sg[m]).wait()
            if q < 2:
                @pl.when(t > 0)
                def _():
                    scatter_drain(m2)
            else:
                scatter_drain(m2)
            if q + 2 < SUP:
                pltpu.async_copy(tab.at[stg_src.at[par].at[q + 2]], rb[m2],
                                 sg[m2])
            else:
                @pl.when(t + 1 < nsa)
                def _():
                    pltpu.async_copy(
                        tab.at[stg_src.at[1 - par].at[q + 2 - SUP]], rb[m2],
                        sg[m2])
            scale_rows(par, q, rb[m])
            pltpu.async_copy(rb[m], acc_sh.at[stg_dst.at[par].at[q]], ss[m],
                             add=True)

        def stage_agg(t, par, sem):
            r = (s + NS * t) * SUP
            pltpu.async_copy(src_hbm.at[pl.ds(r, SUP)], stg_src.at[par], sem)
            pltpu.async_copy(dst_hbm.at[pl.ds(r, SUP)], stg_dst.at[par], sem)
            pltpu.async_copy(ew_hbm.at[pl.ds(r, SUP)], stg_ew.at[par], sem)

        def drain_stage(sem):
            pltpu.make_async_copy(src_hbm.at[pl.ds(0, SUP)], stg_src.at[0],
                                  sem).wait()
            pltpu.make_async_copy(dst_hbm.at[pl.ds(0, SUP)], stg_dst.at[0],
                                  sem).wait()
            pltpu.make_async_copy(ew_hbm.at[pl.ds(0, SUP)], stg_ew.at[0],
                                  sem).wait()

        stage_agg(0, 0, si)
        drain_stage(si)
        pltpu.async_copy(tab.at[stg_src.at[0].at[0]], rb[0], sg[0])
        pltpu.async_copy(tab.at[stg_src.at[0].at[1]], rb[1], sg[1])

        @pl.loop(0, nsa)
        def _(t):
            par = t % 2
            chunk(t, par, 0)
            chunk(t, par, 1)

            @pl.when(t + 1 < nsa)
            def _():
                stage_agg(t + 1, 1 - par, si)

            chunk(t, par, 2)
            chunk(t, par, 3)
            chunk(t, par, 4)
            chunk(t, par, 5)

            @pl.when(t + 1 < nsa)
            def _():
                drain_stage(si)

            chunk(t, par, 6)
            chunk(t, par, 7)

        scatter_drain(2)
        scatter_drain(3)

        plsc.subcore_barrier()

        @pl.loop(s, NSUPZ, step=NS)
        def _(g):
            r0 = g * SUPZ
            pltpu.async_copy(acc_sh.at[pl.ds(r0, SUPZ)],
                             agg_hbm.at[c].at[pl.ds(r0, SUPZ)], sz)

        @pl.loop(s, NSUPZ, step=NS)
        def _(g):
            pltpu.make_async_copy(acc_sh.at[pl.ds(0, SUPZ)],
                                  agg_hbm.at[c].at[pl.ds(0, SUPZ)], sz).wait()

    if first_layer:
        out_type = (jax.ShapeDtypeStruct((NC, N, DH), jnp.float32),
                    jax.ShapeDtypeStruct((N,), jnp.float32))
    else:
        out_type = jax.ShapeDtypeStruct((NC, N, DH), jnp.float32)

    scratch = [
        pltpu.VMEM_SHARED((N, DH), jnp.float32),
    ]
    if first_layer:
        scratch += [
            pltpu.VMEM_SHARED((N,), jnp.float32),
        ]
    scratch += [
        pltpu.VMEM((SUPZ, DH), jnp.float32),
    ]
    if first_layer:
        scratch += [
            pltpu.VMEM((1024,), jnp.float32),
        ]
    scratch += [
        pltpu.VMEM((N,), jnp.float32),
        pltpu.VMEM((2, SUP, CH), jnp.int32),
        pltpu.VMEM((2, SUP, CH), jnp.int32),
        pltpu.VMEM((2, SUP, CH), jnp.float32),
        pltpu.VMEM((CH, DH), jnp.float32),
        pltpu.VMEM((CH, DH), jnp.float32),
        pltpu.VMEM((CH, DH), jnp.float32),
        pltpu.VMEM((CH, DH), jnp.float32),
        pltpu.SemaphoreType.DMA,
        pltpu.SemaphoreType.DMA,
        pltpu.SemaphoreType.DMA,
        pltpu.SemaphoreType.DMA,
        pltpu.SemaphoreType.DMA,
        pltpu.SemaphoreType.DMA,
        pltpu.SemaphoreType.DMA,
        pltpu.SemaphoreType.DMA,
        pltpu.SemaphoreType.DMA,
        pltpu.SemaphoreType.DMA,
    ]
    if first_layer:
        scratch += [
            pltpu.SemaphoreType.DMA,
        ]

    mesh = plsc.VectorSubcoreMesh(core_axis_name="c", subcore_axis_name="s",
                                  num_cores=NC, num_subcores=NS)
    cp = pltpu.CompilerParams()
    if "needs_layout_passes" in pltpu.CompilerParams.__dataclass_fields__:
        cp = dataclasses.replace(cp, needs_layout_passes=False)
    cp = dataclasses.replace(cp, use_tc_tiling_on_sc=False)
    return pl.kernel(body, out_type=out_type, mesh=mesh, scratch_types=scratch,
                     compiler_params=cp)


def _tc_layer1(agg, x, dinv2d, W1, b1r):
    BR = 1000

    def body(a_ref, x_ref, dv_ref, w_ref, b_ref, o_ref):
        dv = dv_ref[...]
        dv2 = dv * dv
        xb = x_ref[...]
        uL = dv * a_ref[0] + dv2 * xb[:, :DH]
        uR = dv * a_ref[1] + dv2 * xb[:, DH:]
        y = (jnp.dot(uL, w_ref[pl.ds(0, DH), :],
                     preferred_element_type=jnp.float32)
             + jnp.dot(uR, w_ref[pl.ds(DH, DH), :],
                       preferred_element_type=jnp.float32))
        y = jnp.maximum(y + b_ref[...], 0.0)
        o_ref[0] = y[:, :DH]
        o_ref[1] = y[:, DH:]

    return pl.pallas_call(
        body,
        grid=(N // BR,),
        in_specs=[
            pl.BlockSpec((NC, BR, DH), lambda i: (0, i, 0)),
            pl.BlockSpec((BR, D), lambda i: (i, 0)),
            pl.BlockSpec((BR, 1), lambda i: (i, 0)),
            pl.BlockSpec((D, D), lambda i: (0, 0)),
            pl.BlockSpec((1, D), lambda i: (0, 0)),
        ],
        out_specs=pl.BlockSpec((NC, BR, DH), lambda i: (0, i, 0)),
        out_shape=jax.ShapeDtypeStruct((NC, N, DH), jnp.float32),
    )(agg, x, dinv2d, W1, b1r)


def _tc_layer2(agg, y1s, dinv2d, W2, b2r, Wfc, bfcr):
    BR = 1000

    def body(a_ref, y_ref, dv_ref, w2_ref, b2_ref, wf_ref, bf_ref, o_ref):
        dv = dv_ref[...]
        dv2 = dv * dv
        uL = dv * a_ref[0] + dv2 * y_ref[0]
        uR = dv * a_ref[1] + dv2 * y_ref[1]
        y2 = (jnp.dot(uL, w2_ref[pl.ds(0, DH), :],
                      preferred_element_type=jnp.float32)
              + jnp.dot(uR, w2_ref[pl.ds(DH, DH), :],
                        preferred_element_type=jnp.float32))
        y2 = jnp.maximum(y2 + b2_ref[...], 0.0)
        o_ref[...] = (jnp.dot(y2, wf_ref[...],
                              preferred_element_type=jnp.float32)
                      + bf_ref[...])

    return pl.pallas_call(
        body,
        grid=(N // BR,),
        in_specs=[
            pl.BlockSpec((NC, BR, DH), lambda i: (0, i, 0)),
            pl.BlockSpec((NC, BR, DH), lambda i: (0, i, 0)),
            pl.BlockSpec((BR, 1), lambda i: (i, 0)),
            pl.BlockSpec((D, D), lambda i: (0, 0)),
            pl.BlockSpec((1, D), lambda i: (0, 0)),
            pl.BlockSpec((D, DO), lambda i: (0, 0)),
            pl.BlockSpec((1, DO), lambda i: (0, 0)),
        ],
        out_specs=pl.BlockSpec((BR, DO), lambda i: (i, 0)),
        out_shape=jax.ShapeDtypeStruct((N, DO), jnp.float32),
    )(agg, y1s, dinv2d, W2, b2r, Wfc, bfcr)


@functools.lru_cache(maxsize=None)
def _sc_agg(first_layer):
    return _make_sc_agg(first_layer)


def kernel(x, edge_index, edge_weight, W1, b1, W2, b2, Wfc, bfc):
    src2 = edge_index[0].reshape(RC, CH)
    dst2 = edge_index[1].reshape(RC, CH)
    ew2 = edge_weight.reshape(RC, CH)
    xs = jnp.stack([x[:, :DH], x[:, DH:]])

    agg1, dinv = _sc_agg(True)(src2, dst2, ew2, xs)
    dinv2d = dinv.reshape(N, 1)
    y1s = _tc_layer1(agg1, x, dinv2d, W1, b1.reshape(1, D))
    agg2 = _sc_agg(False)(src2, dst2, ew2, y1s, dinv)
    out = _tc_layer2(agg2, y1s, dinv2d, W2, b2.reshape(1, D),
                     Wfc, bfc.reshape(1, DO))
    return out


# speedup vs baseline: 26.2310x; 1.0392x over previous
import dataclasses
import functools

import jax
import jax.numpy as jnp
from jax import lax
from jax.experimental import pallas as pl
from jax.experimental.pallas import tpu as pltpu
from jax.experimental.pallas import tpu_sc as plsc

N = 10000
E = 320000
D = 128
DH = 64
DO = 16
CH = 128
RC = E // CH
SUP = 4
NSUP = RC // SUP
SUPZ = 40
NSUPZ = N // SUPZ
NC, NS = 2, 16


def _fast_rsqrt(v):
    b = plsc.bitcast(v, jnp.int32)
    y = plsc.bitcast(jnp.int32(0x5F3759DF) - lax.shift_right_logical(b, 1),
                     jnp.float32)
    y = y * (1.5 - 0.5 * v * y * y)
    y = y * (1.5 - 0.5 * v * y * y)
    y = y * (1.5 - 0.5 * v * y * y)
    return y


def _make_sc_agg(first_layer):
    def body(*refs):
        if first_layer:
            (src_hbm, dst_hbm, ew_hbm, tabs_hbm,
             agg_hbm, dinv_hbm,
             acc_sh, deg_sh, zbuf, z1buf, dinv_t,
             stg_src, stg_dst, stg_ew, rb0, rb1, rb2, rb3,
             sz, si, sg0, sg1, sg2, sg3, ss0, ss1, ss2, ss3, sdeg) = refs
        else:
            (src_hbm, dst_hbm, ew_hbm, tabs_hbm, dinv_in,
             agg_hbm,
             acc_sh, zbuf, dinv_t,
             stg_src, stg_dst, stg_ew, rb0, rb1, rb2, rb3,
             sz, si, sg0, sg1, sg2, sg3, ss0, ss1, ss2, ss3) = refs
        c = lax.axis_index("c")
        s = lax.axis_index("s")
        rb = (rb0, rb1, rb2, rb3)
        sg = (sg0, sg1, sg2, sg3)
        ss = (ss0, ss1, ss2, ss3)
        tab = tabs_hbm.at[c]

        nsa = (NSUP + NS - 1 - s) // NS

        @pl.loop(0, SUPZ)
        def _(r):
            for j in range(0, DH, 16):
                zbuf[r, pl.ds(j, 16)] = jnp.zeros((16,), jnp.float32)

        @pl.loop(s, NSUPZ, step=NS)
        def _(g):
            pltpu.async_copy(zbuf, acc_sh.at[pl.ds(g * SUPZ, SUPZ)], sz)

        if first_layer:
            @pl.loop(0, 1024, step=16)
            def _(i):
                z1buf[pl.ds(i, 16)] = jnp.zeros((16,), jnp.float32)

            @pl.when(s < 10)
            def _():
                pltpu.sync_copy(z1buf.at[pl.ds(0, 1000)],
                                deg_sh.at[pl.ds(s * 1000, 1000)])

        @pl.loop(s, NSUPZ, step=NS)
        def _(g):
            pltpu.make_async_copy(zbuf, acc_sh.at[pl.ds(0, SUPZ)], sz).wait()

        plsc.subcore_barrier()

        if first_layer:
            pltpu.async_copy(dst_hbm.at[pl.ds(s * SUP, SUP)], stg_dst.at[0],
                             si)
            pltpu.async_copy(ew_hbm.at[pl.ds(s * SUP, SUP)], stg_ew.at[0], si)

            @pl.loop(0, nsa)
            def _(t):
                par = t % 2

                @pl.when(t > 0)
                def _():
                    @pl.loop(0, SUP)
                    def _(j):
                        pltpu.make_async_copy(
                            stg_ew.at[0].at[0],
                            deg_sh.at[stg_dst.at[0].at[0]], sdeg).wait()

                pltpu.make_async_copy(dst_hbm.at[pl.ds(0, SUP)],
                                      stg_dst.at[0], si).wait()
                pltpu.make_async_copy(ew_hbm.at[pl.ds(0, SUP)],
                                      stg_ew.at[0], si).wait()

                @pl.when(t + 1 < nsa)
                def _():
                    r1 = (s + NS * (t + 1)) * SUP
                    pltpu.async_copy(dst_hbm.at[pl.ds(r1, SUP)],
                                     stg_dst.at[1 - par], si)
                    pltpu.async_copy(ew_hbm.at[pl.ds(r1, SUP)],
                                     stg_ew.at[1 - par], si)

                @pl.loop(0, SUP)
                def _(j):
                    pltpu.async_copy(stg_ew.at[par].at[j],
                                     deg_sh.at[stg_dst.at[par].at[j]], sdeg,
                                     add=True)

            @pl.loop(0, SUP)
            def _(j):
                pltpu.make_async_copy(stg_ew.at[0].at[0],
                                      deg_sh.at[stg_dst.at[0].at[0]],
                                      sdeg).wait()

            plsc.subcore_barrier()

            pltpu.sync_copy(deg_sh, dinv_t)

            @pl.loop(0, N, step=16)
            def _(i):
                v = dinv_t[pl.ds(i, 16)] + 1.0
                dinv_t[pl.ds(i, 16)] = _fast_rsqrt(v)

            @pl.when(jnp.logical_and(c == 0, s < 10))
            def _():
                pltpu.sync_copy(dinv_t.at[pl.ds(s * 1000, 1000)],
                                dinv_hbm.at[pl.ds(s * 1000, 1000)])
        else:
            pltpu.sync_copy(dinv_in, dinv_t)

        def scale_rows(par, q, rbm):
            @pl.loop(0, CH, step=16)
            def _(i):
                idx = stg_src[par, q, pl.ds(i, 16)]
                dv = plsc.load_gather(dinv_t, [idx])
                wv = stg_ew[par, q, pl.ds(i, 16)] * dv
                for l in range(16):
                    wl = wv[l]
                    for jj in range(0, DH, 16):
                        rbm[i + l, pl.ds(jj, 16)] = (
                            rbm[i + l, pl.ds(jj, 16)] * wl)

        def scatter_drain(m):
            pltpu.make_async_copy(rb[m], acc_sh.at[stg_dst.at[0].at[0]],
                                  ss[m]).wait()

        def chunk(t, par, q):
            m = q % 4
            m2 = (q + 2) % 4
            pltpu.make_async_copy(tab.at[stg_src.at[par].at[q]], rb[m],
                                  sg[m]).wait()
            if q < 2:
                @pl.when(t > 0)
                def _():
                    scatter_drain(m2)
            else:
                scatter_drain(m2)
            if q + 2 < SUP:
                pltpu.async_copy(tab.at[stg_src.at[par].at[q + 2]], rb[m2],
                                 sg[m2])
            else:
                @pl.when(t + 1 < nsa)
                def _():
                    pltpu.async_copy(
                        tab.at[stg_src.at[(par + 1) % 3].at[q + 2 - SUP]],
                        rb[m2], sg[m2])
            scale_rows(par, q, rb[m])
            pltpu.async_copy(rb[m], acc_sh.at[stg_dst.at[par].at[q]], ss[m],
                             add=True)

        def stage_agg(t, par, sem):
            r = (s + NS * t) * SUP
            pltpu.async_copy(src_hbm.at[pl.ds(r, SUP)], stg_src.at[par], sem)
            pltpu.async_copy(dst_hbm.at[pl.ds(r, SUP)], stg_dst.at[par], sem)
            pltpu.async_copy(ew_hbm.at[pl.ds(r, SUP)], stg_ew.at[par], sem)

        def drain_stage(sem):
            pltpu.make_async_copy(src_hbm.at[pl.ds(0, SUP)], stg_src.at[0],
                                  sem).wait()
            pltpu.make_async_copy(dst_hbm.at[pl.ds(0, SUP)], stg_dst.at[0],
                                  sem).wait()
            pltpu.make_async_copy(ew_hbm.at[pl.ds(0, SUP)], stg_ew.at[0],
                                  sem).wait()

        stage_agg(0, 0, si)
        drain_stage(si)
        stage_agg(1, 1, si)
        pltpu.async_copy(tab.at[stg_src.at[0].at[0]], rb[0], sg[0])
        pltpu.async_copy(tab.at[stg_src.at[0].at[1]], rb[1], sg[1])

        @pl.loop(0, nsa)
        def _(t):
            par = t % 3
            chunk(t, par, 0)
            chunk(t, par, 1)

            @pl.when(t + 1 < nsa)
            def _():
                drain_stage(si)

            @pl.when(t + 2 < nsa)
            def _():
                stage_agg(t + 2, (t + 2) % 3, si)

            chunk(t, par, 2)
            chunk(t, par, 3)

        scatter_drain(2)
        scatter_drain(3)

        plsc.subcore_barrier()

        @pl.loop(s, NSUPZ, step=NS)
        def _(g):
            r0 = g * SUPZ
            pltpu.async_copy(acc_sh.at[pl.ds(r0, SUPZ)],
                             agg_hbm.at[c].at[pl.ds(r0, SUPZ)], sz)

        @pl.loop(s, NSUPZ, step=NS)
        def _(g):
            pltpu.make_async_copy(acc_sh.at[pl.ds(0, SUPZ)],
                                  agg_hbm.at[c].at[pl.ds(0, SUPZ)], sz).wait()

    if first_layer:
        out_type = (jax.ShapeDtypeStruct((NC, N, DH), jnp.float32),
                    jax.ShapeDtypeStruct((N,), jnp.float32))
    else:
        out_type = jax.ShapeDtypeStruct((NC, N, DH), jnp.float32)

    scratch = [
        pltpu.VMEM_SHARED((N, DH), jnp.float32),
    ]
    if first_layer:
        scratch += [
            pltpu.VMEM_SHARED((N,), jnp.float32),
        ]
    scratch += [
        pltpu.VMEM((SUPZ, DH), jnp.float32),
    ]
    if first_layer:
        scratch += [
            pltpu.VMEM((1024,), jnp.float32),
        ]
    scratch += [
        pltpu.VMEM((N,), jnp.float32),
        pltpu.VMEM((3, SUP, CH), jnp.int32),
        pltpu.VMEM((3, SUP, CH), jnp.int32),
        pltpu.VMEM((3, SUP, CH), jnp.float32),
        pltpu.VMEM((CH, DH), jnp.float32),
        pltpu.VMEM((CH, DH), jnp.float32),
        pltpu.VMEM((CH, DH), jnp.float32),
        pltpu.VMEM((CH, DH), jnp.float32),
        pltpu.SemaphoreType.DMA,
        pltpu.SemaphoreType.DMA,
        pltpu.SemaphoreType.DMA,
        pltpu.SemaphoreType.DMA,
        pltpu.SemaphoreType.DMA,
        pltpu.SemaphoreType.DMA,
        pltpu.SemaphoreType.DMA,
        pltpu.SemaphoreType.DMA,
        pltpu.SemaphoreType.DMA,
        pltpu.SemaphoreType.DMA,
    ]
    if first_layer:
        scratch += [
            pltpu.SemaphoreType.DMA,
        ]

    mesh = plsc.VectorSubcoreMesh(core_axis_name="c", subcore_axis_name="s",
                                  num_cores=NC, num_subcores=NS)
    cp = pltpu.CompilerParams()
    if "needs_layout_passes" in pltpu.CompilerParams.__dataclass_fields__:
        cp = dataclasses.replace(cp, needs_layout_passes=False)
    cp = dataclasses.replace(cp, use_tc_tiling_on_sc=False)
    return pl.kernel(body, out_type=out_type, mesh=mesh, scratch_types=scratch,
                     compiler_params=cp)


def _tc_layer1(agg, x, dinv2d, W1, b1r):
    BR = 2000

    def body(a_ref, x_ref, dv_ref, w_ref, b_ref, o_ref):
        dv = dv_ref[...]
        dv2 = dv * dv
        xb = x_ref[...]
        uL = dv * a_ref[0] + dv2 * xb[:, :DH]
        uR = dv * a_ref[1] + dv2 * xb[:, DH:]
        y = (jnp.dot(uL, w_ref[pl.ds(0, DH), :],
                     preferred_element_type=jnp.float32)
             + jnp.dot(uR, w_ref[pl.ds(DH, DH), :],
                       preferred_element_type=jnp.float32))
        y = jnp.maximum(y + b_ref[...], 0.0)
        o_ref[0] = y[:, :DH]
        o_ref[1] = y[:, DH:]

    return pl.pallas_call(
        body,
        grid=(N // BR,),
        in_specs=[
            pl.BlockSpec((NC, BR, DH), lambda i: (0, i, 0)),
            pl.BlockSpec((BR, D), lambda i: (i, 0)),
            pl.BlockSpec((BR, 1), lambda i: (i, 0)),
            pl.BlockSpec((D, D), lambda i: (0, 0)),
            pl.BlockSpec((1, D), lambda i: (0, 0)),
        ],
        out_specs=pl.BlockSpec((NC, BR, DH), lambda i: (0, i, 0)),
        out_shape=jax.ShapeDtypeStruct((NC, N, DH), jnp.float32),
    )(agg, x, dinv2d, W1, b1r)


def _tc_layer2(agg, y1s, dinv2d, W2, b2r, Wfc, bfcr):
    BR = 2000

    def body(a_ref, y_ref, dv_ref, w2_ref, b2_ref, wf_ref, bf_ref, o_ref):
        dv = dv_ref[...]
        dv2 = dv * dv
        uL = dv * a_ref[0] + dv2 * y_ref[0]
        uR = dv * a_ref[1] + dv2 * y_ref[1]
        y2 = (jnp.dot(uL, w2_ref[pl.ds(0, DH), :],
                      preferred_element_type=jnp.float32)
              + jnp.dot(uR, w2_ref[pl.ds(DH, DH), :],
                        preferred_element_type=jnp.float32))
        y2 = jnp.maximum(y2 + b2_ref[...], 0.0)
        o_ref[...] = (jnp.dot(y2, wf_ref[...],
                              preferred_element_type=jnp.float32)
                      + bf_ref[...])

    return pl.pallas_call(
        body,
        grid=(N // BR,),
        in_specs=[
            pl.BlockSpec((NC, BR, DH), lambda i: (0, i, 0)),
            pl.BlockSpec((NC, BR, DH), lambda i: (0, i, 0)),
            pl.BlockSpec((BR, 1), lambda i: (i, 0)),
            pl.BlockSpec((D, D), lambda i: (0, 0)),
            pl.BlockSpec((1, D), lambda i: (0, 0)),
            pl.BlockSpec((D, DO), lambda i: (0, 0)),
            pl.BlockSpec((1, DO), lambda i: (0, 0)),
        ],
        out_specs=pl.BlockSpec((BR, DO), lambda i: (i, 0)),
        out_shape=jax.ShapeDtypeStruct((N, DO), jnp.float32),
    )(agg, y1s, dinv2d, W2, b2r, Wfc, bfcr)


@functools.lru_cache(maxsize=None)
def _sc_agg(first_layer):
    return _make_sc_agg(first_layer)


def kernel(x, edge_index, edge_weight, W1, b1, W2, b2, Wfc, bfc):
    src2 = edge_index[0].reshape(RC, CH)
    dst2 = edge_index[1].reshape(RC, CH)
    ew2 = edge_weight.reshape(RC, CH)
    xs = jnp.stack([x[:, :DH], x[:, DH:]])

    agg1, dinv = _sc_agg(True)(src2, dst2, ew2, xs)
    dinv2d = dinv.reshape(N, 1)
    y1s = _tc_layer1(agg1, x, dinv2d, W1, b1.reshape(1, D))
    agg2 = _sc_agg(False)(src2, dst2, ew2, y1s, dinv)
    out = _tc_layer2(agg2, y1s, dinv2d, W2, b2.reshape(1, D),
                     Wfc, bfc.reshape(1, DO))
    return out
